# baseline (speedup 1.0000x reference)
import jax
import jax.numpy as jnp
from jax.experimental import pallas as pl
from jax.experimental.pallas import tpu as pltpu

N_TOKENS = 16384
EMBED_DIM = 2048
LOW_RANK_DIM = 16
OUT_DIM = 64

BLOCK_T = 2048


def _body(x_hbm, w1t_ref, w2t_ref, out_hbm):
    w1t = w1t_ref[...]
    w2t = w2t_ref[...]

    def step(x_ref, out_ref):
        h = jnp.dot(x_ref[...], w1t, preferred_element_type=jnp.float32)
        out_ref[...] = jnp.dot(h, w2t, preferred_element_type=jnp.float32)

    pipeline = pltpu.emit_pipeline(
        step,
        grid=(N_TOKENS // BLOCK_T,),
        in_specs=[
            pl.BlockSpec((BLOCK_T, EMBED_DIM), lambda i: (i, 0),
                         pipeline_mode=pl.Buffered(buffer_count=3,
                                                   use_lookahead=True)),
        ],
        out_specs=[
            pl.BlockSpec((BLOCK_T, OUT_DIM), lambda i: (i, 0)),
        ],
    )
    pipeline(x_hbm, out_hbm)


def kernel(x, w1, w2):
    n = x.shape[0]
    w1t = w1.T
    w2t = w2.T
    return pl.pallas_call(
        _body,
        in_specs=[
            pl.BlockSpec(memory_space=pl.ANY),
            pl.BlockSpec(memory_space=pltpu.MemorySpace.VMEM),
            pl.BlockSpec(memory_space=pltpu.MemorySpace.VMEM),
        ],
        out_specs=pl.BlockSpec(memory_space=pl.ANY),
        out_shape=jax.ShapeDtypeStruct((n, OUT_DIM), jnp.float32),
    )(x, w1t, w2t)


# speedup vs baseline: 1.0292x; 1.0292x over previous
import jax
import jax.numpy as jnp
from jax.experimental import pallas as pl
from jax.experimental.pallas import tpu as pltpu

N_TOKENS = 16384
EMBED_DIM = 2048
LOW_RANK_DIM = 16
OUT_DIM = 64

BLOCK_T = 2048


def _fused_body(x_ref, w1t_ref, w2t_ref, out_ref):
    h = jnp.dot(x_ref[...], w1t_ref[...], preferred_element_type=jnp.float32)
    out_ref[...] = jnp.dot(h, w2t_ref[...], preferred_element_type=jnp.float32)


def kernel(x, w1, w2):
    n = x.shape[0]
    w1t = w1.T
    w2t = w2.T
    grid = (n // BLOCK_T,)
    return pl.pallas_call(
        _fused_body,
        grid=grid,
        in_specs=[
            pl.BlockSpec((BLOCK_T, EMBED_DIM), lambda i: (i, 0)),
            pl.BlockSpec((EMBED_DIM, LOW_RANK_DIM), lambda i: (0, 0)),
            pl.BlockSpec((LOW_RANK_DIM, OUT_DIM), lambda i: (0, 0)),
        ],
        out_specs=pl.BlockSpec((BLOCK_T, OUT_DIM), lambda i: (i, 0)),
        out_shape=jax.ShapeDtypeStruct((n, OUT_DIM), jnp.float32),
        compiler_params=pltpu.CompilerParams(
            dimension_semantics=("arbitrary",),
        ),
    )(x, w1t, w2t)
